# baseline (speedup 1.0000x reference)
import functools

import jax
import jax.numpy as jnp
from jax import lax
from jax.experimental import pallas as pl
from jax.experimental.pallas import tpu as pltpu
from jax.experimental.pallas import tpu_sc as plsc

N = 10000
E = 160000
D = 256
DH = D // 2
NUM_ITERS = 4
GAMMA = 0.1
EPS = 0.1

NTILES = 16
NCORES = 2
CH = 128
N_PAD = 10240
RPT = N_PAD // NTILES
PROW = 128
NPIECE = N_PAD // PROW
PPT = NPIECE // NTILES
WN = 1024
E_ALLOC = 163840

ROW_BLK = 1000
GRID = N // ROW_BLK

_dn = (((1,), (1,)), ((), ()))


def _mm(a, b):
    return lax.dot_general(a, b, _dn, preferred_element_type=jnp.float32)


def _tc_pro_body(x_ref, we_ref, be_ref, wl_ref, a_ref, h_ref, nb_ref, ha_ref):
    hb = _mm(x_ref[...], we_ref[...]) + be_ref[...]
    h_ref[...] = hb
    nb = _mm(hb, wl_ref[...])
    nb_ref[0] = nb[:, :DH]
    nb_ref[1] = nb[:, DH:]
    ha_ref[...] = _mm(hb, a_ref[...])


def _tc_upd_body(h_ref, ha_ref, agg_ref, wl_ref, a_ref, hn_ref, nb_ref, han_ref):
    conv = ha_ref[...] + jnp.concatenate([agg_ref[0], agg_ref[1]], axis=1)
    hn = h_ref[...] + EPS * jnp.tanh(conv)
    hn_ref[...] = hn
    nb = _mm(hn, wl_ref[...])
    nb_ref[0] = nb[:, :DH]
    nb_ref[1] = nb[:, DH:]
    han_ref[...] = _mm(hn, a_ref[...])


def _tc_ro_body(h_ref, ha_ref, agg_ref, wro_ref, bro_ref, out_ref):
    conv = ha_ref[...] + jnp.concatenate([agg_ref[0], agg_ref[1]], axis=1)
    hn = h_ref[...] + EPS * jnp.tanh(conv)
    out_ref[...] = _mm(hn, wro_ref[...]) + bro_ref[...]


_row_spec = pl.BlockSpec((ROW_BLK, D), lambda i: (i, 0))
_stk_spec = pl.BlockSpec((2, ROW_BLK, DH), lambda i: (0, i, 0))
_w_spec = pl.BlockSpec((D, D), lambda i: (0, 0))
_b_spec = pl.BlockSpec((1, D), lambda i: (0, 0))

_f32 = jnp.float32
_sds = jax.ShapeDtypeStruct

_tc_pro = pl.pallas_call(
    _tc_pro_body,
    grid=(GRID,),
    in_specs=[_row_spec, _w_spec, _b_spec, _w_spec, _w_spec],
    out_specs=[_row_spec, _stk_spec, _row_spec],
    out_shape=[_sds((N, D), _f32), _sds((2, N, DH), _f32), _sds((N, D), _f32)],
)

_tc_upd = pl.pallas_call(
    _tc_upd_body,
    grid=(GRID,),
    in_specs=[_row_spec, _row_spec, _stk_spec, _w_spec, _w_spec],
    out_specs=[_row_spec, _stk_spec, _row_spec],
    out_shape=[_sds((N, D), _f32), _sds((2, N, DH), _f32), _sds((N, D), _f32)],
)

_tc_ro = pl.pallas_call(
    _tc_ro_body,
    grid=(GRID,),
    in_specs=[_row_spec, _row_spec, _stk_spec, _w_spec, _b_spec],
    out_specs=_row_spec,
    out_shape=_sds((N, D), _f32),
)


def _sc_body(table, srcp, dstp, wp, zeros, sinfo, agg_out,
             sv, src_w, dst_w, w_w, wbuf, sbuf, dst_chunk, acc):
    c = lax.axis_index("c")
    s = lax.axis_index("s")
    out_off = c * N_PAD
    i16 = lax.iota(jnp.int32, 16)

    pltpu.sync_copy(sinfo.at[s], sv)
    bounds = sv[0]

    pltpu.sync_copy(zeros.at[pl.ds(s * RPT, RPT)], acc.at[pl.ds(s * RPT, RPT)])
    plsc.subcore_barrier()

    def piece_body(q, pbs, pe):
        a0 = jnp.bitwise_and(pbs, jnp.int32(-256))
        nwin = lax.shift_right_logical(pe - a0 + (WN - 1), 10)
        prow = (s * PPT + q) * PROW

        @pl.when(nwin > 0)
        def _():
            toff = pl.multiple_of(c * N + prow, 8)
            pltpu.sync_copy(table.at[pl.ds(toff, PROW)], wbuf)

            def win_body(k, carry2):
                wbase = pl.multiple_of(a0 + k * WN, 256)
                pltpu.sync_copy(srcp.at[pl.ds(wbase, WN)], src_w)
                pltpu.sync_copy(dstp.at[pl.ds(wbase, WN)], dst_w)
                pltpu.sync_copy(wp.at[pl.ds(wbase, WN)], w_w)

                def chunk_body(m, carry3):
                    def grp(g, carry4):
                        eb = m * CH + g * 16
                        sl16 = pl.ds(eb, 16)
                        sv16 = src_w[sl16]
                        rl16 = jnp.minimum(
                            jnp.maximum(sv16 - prow, 0), PROW - 1)
                        eidx = wbase + eb + i16
                        msk = (eidx >= pbs) & (eidx < pe)
                        wm16 = jnp.where(msk, w_w[sl16], 0.0)
                        dst_chunk[pl.ds(g * 16, 16)] = dst_w[sl16]
                        for lane in range(16):
                            rloc = rl16[lane]
                            wv = wm16[lane]
                            er = g * 16 + lane
                            for j in range(DH // 16):
                                sl = pl.ds(j * 16, 16)
                                sbuf[er, sl] = wbuf[rloc, sl] * wv
                        return carry4

                    lax.fori_loop(0, CH // 16, grp, 0)
                    pltpu.sync_copy(sbuf, acc.at[dst_chunk], add=True)
                    return carry3

                lax.fori_loop(0, WN // CH, chunk_body, 0)
                return carry2

            lax.fori_loop(0, nwin, win_body, 0)

    for q in range(PPT):
        piece_body(q, bounds[q], bounds[q + 1])
    plsc.subcore_barrier()

    pltpu.sync_copy(acc.at[pl.ds(s * RPT, RPT)],
                    agg_out.at[pl.ds(out_off + s * RPT, RPT)])


_sc_agg = functools.partial(
    pl.kernel,
    out_type=_sds((2 * N_PAD, DH), _f32),
    mesh=plsc.VectorSubcoreMesh(core_axis_name="c", subcore_axis_name="s"),
    scratch_types=[
        pltpu.VMEM((1, 16), jnp.int32),
        pltpu.VMEM((WN,), jnp.int32),
        pltpu.VMEM((WN,), jnp.int32),
        pltpu.VMEM((WN,), _f32),
        pltpu.VMEM((PROW, DH), _f32),
        pltpu.VMEM((CH, DH), _f32),
        pltpu.VMEM((CH,), jnp.int32),
        pltpu.VMEM_SHARED((N_PAD, DH), _f32),
    ],
)(_sc_body)


def kernel(x, edge_index, edge_weight, W_emb, b_emb, W, W_lin, W_ro, b_ro):
    A = W - W.T - GAMMA * jnp.eye(D, dtype=W.dtype)
    order = jnp.argsort(edge_index[0])
    src = edge_index[0].astype(jnp.int32)[order]
    dst = edge_index[1].astype(jnp.int32)[order]
    w = edge_weight.astype(jnp.float32)[order]
    pad = E_ALLOC - E
    srcp = jnp.pad(src, (0, pad), constant_values=1 << 20)
    dstp = jnp.pad(dst, (0, pad))
    wp = jnp.pad(w, (0, pad))
    pb = jnp.searchsorted(
        srcp, jnp.arange(NPIECE + 1, dtype=jnp.int32) * PROW).astype(jnp.int32)
    tidx = jnp.arange(NTILES)[:, None] * PPT + jnp.arange(PPT + 1)[None, :]
    sinfo = jnp.zeros((NTILES, 1, 16), jnp.int32).at[:, 0, :PPT + 1].set(
        pb[tidx])
    zeros = jnp.zeros((N_PAD, DH), _f32)
    be = b_emb.reshape(1, D)
    bro = b_ro.reshape(1, D)

    h, nb, ha = _tc_pro(x, W_emb, be, W_lin, A)
    for _ in range(NUM_ITERS - 1):
        agg = _sc_agg(nb.reshape(2 * N, DH), srcp, dstp, wp, zeros, sinfo)
        h, nb, ha = _tc_upd(h, ha, agg.reshape(2, N_PAD, DH), W_lin, A)
    agg = _sc_agg(nb.reshape(2 * N, DH), srcp, dstp, wp, zeros, sinfo)
    out = _tc_ro(h, ha, agg.reshape(2, N_PAD, DH), W_ro, bro)
    return out


# speedup vs baseline: 2.2125x; 2.2125x over previous
import functools

import jax
import jax.numpy as jnp
from jax import lax
from jax.experimental import pallas as pl
from jax.experimental.pallas import tpu as pltpu
from jax.experimental.pallas import tpu_sc as plsc

N = 10000
E = 160000
D = 256
DH = D // 2
NUM_ITERS = 4
GAMMA = 0.1
EPS = 0.1

NTILES = 16
NCORES = 2
CH = 128
NCH = -(-E // (NTILES * CH))
EPT = NCH * CH
E_PAD = NTILES * EPT
N_PAD = 10240
RPT = N_PAD // NTILES

ROW_BLK = 1000
GRID = N // ROW_BLK

_dn = (((1,), (1,)), ((), ()))


def _mm(a, b):
    return lax.dot_general(a, b, _dn, preferred_element_type=jnp.float32)


def _tc_pro_body(x_ref, we_ref, be_ref, wl_ref, a_ref, h_ref, nb_ref, ha_ref):
    hb = _mm(x_ref[...], we_ref[...]) + be_ref[...]
    h_ref[...] = hb
    nb = _mm(hb, wl_ref[...])
    nb_ref[0] = nb[:, :DH]
    nb_ref[1] = nb[:, DH:]
    ha_ref[...] = _mm(hb, a_ref[...])


def _tc_upd_body(h_ref, ha_ref, agg_ref, wl_ref, a_ref, hn_ref, nb_ref, han_ref):
    conv = ha_ref[...] + jnp.concatenate([agg_ref[0], agg_ref[1]], axis=1)
    hn = h_ref[...] + EPS * jnp.tanh(conv)
    hn_ref[...] = hn
    nb = _mm(hn, wl_ref[...])
    nb_ref[0] = nb[:, :DH]
    nb_ref[1] = nb[:, DH:]
    han_ref[...] = _mm(hn, a_ref[...])


def _tc_ro_body(h_ref, ha_ref, agg_ref, wro_ref, bro_ref, out_ref):
    conv = ha_ref[...] + jnp.concatenate([agg_ref[0], agg_ref[1]], axis=1)
    hn = h_ref[...] + EPS * jnp.tanh(conv)
    out_ref[...] = _mm(hn, wro_ref[...]) + bro_ref[...]


_row_spec = pl.BlockSpec((ROW_BLK, D), lambda i: (i, 0))
_stk_spec = pl.BlockSpec((2, ROW_BLK, DH), lambda i: (0, i, 0))
_w_spec = pl.BlockSpec((D, D), lambda i: (0, 0))
_b_spec = pl.BlockSpec((1, D), lambda i: (0, 0))

_f32 = jnp.float32
_sds = jax.ShapeDtypeStruct

_tc_pro = pl.pallas_call(
    _tc_pro_body,
    grid=(GRID,),
    in_specs=[_row_spec, _w_spec, _b_spec, _w_spec, _w_spec],
    out_specs=[_row_spec, _stk_spec, _row_spec],
    out_shape=[_sds((N, D), _f32), _sds((2, N, DH), _f32), _sds((N, D), _f32)],
)

_tc_upd = pl.pallas_call(
    _tc_upd_body,
    grid=(GRID,),
    in_specs=[_row_spec, _row_spec, _stk_spec, _w_spec, _w_spec],
    out_specs=[_row_spec, _stk_spec, _row_spec],
    out_shape=[_sds((N, D), _f32), _sds((2, N, DH), _f32), _sds((N, D), _f32)],
)

_tc_ro = pl.pallas_call(
    _tc_ro_body,
    grid=(GRID,),
    in_specs=[_row_spec, _row_spec, _stk_spec, _w_spec, _b_spec],
    out_specs=_row_spec,
    out_shape=_sds((N, D), _f32),
)


def _sc_body(table, srcp, dstp, wp, zeros, agg_out,
             src_all, dst_all, w_all, src_chunk, dst_chunk, gath, acc):
    c = lax.axis_index("c")
    s = lax.axis_index("s")
    off = c * N
    out_off = c * N_PAD

    pltpu.sync_copy(srcp.at[s], src_all)
    pltpu.sync_copy(dstp.at[s], dst_all)
    pltpu.sync_copy(wp.at[s], w_all)

    pltpu.sync_copy(zeros.at[pl.ds(s * RPT, RPT)], acc.at[pl.ds(s * RPT, RPT)])
    plsc.subcore_barrier()

    def chunk_body(cs, carry):
        for j in range(CH // 16):
            sl = pl.ds(j * 16, 16)
            src_chunk[sl] = src_all[cs, sl] + off
            dst_chunk[sl] = dst_all[cs, sl]
        pltpu.sync_copy(table.at[src_chunk], gath)

        def scale(g, carry2):
            wv16 = w_all[cs, pl.ds(g * 16, 16)]
            for lane in range(16):
                wv = wv16[lane]
                e = g * 16 + lane
                for j in range(DH // 16):
                    sl = pl.ds(j * 16, 16)
                    gath[e, sl] = gath[e, sl] * wv
            return carry2

        lax.fori_loop(0, CH // 16, scale, 0)
        pltpu.sync_copy(gath, acc.at[dst_chunk], add=True)
        return carry

    lax.fori_loop(0, NCH, chunk_body, 0)
    plsc.subcore_barrier()

    pltpu.sync_copy(acc.at[pl.ds(s * RPT, RPT)],
                    agg_out.at[pl.ds(out_off + s * RPT, RPT)])


_sc_agg = functools.partial(
    pl.kernel,
    out_type=_sds((2 * N_PAD, DH), _f32),
    mesh=plsc.VectorSubcoreMesh(core_axis_name="c", subcore_axis_name="s"),
    scratch_types=[
        pltpu.VMEM((NCH, CH), jnp.int32),
        pltpu.VMEM((NCH, CH), jnp.int32),
        pltpu.VMEM((NCH, CH), _f32),
        pltpu.VMEM((CH,), jnp.int32),
        pltpu.VMEM((CH,), jnp.int32),
        pltpu.VMEM((CH, DH), _f32),
        pltpu.VMEM_SHARED((N_PAD, DH), _f32),
    ],
)(_sc_body)


def kernel(x, edge_index, edge_weight, W_emb, b_emb, W, W_lin, W_ro, b_ro):
    A = W - W.T - GAMMA * jnp.eye(D, dtype=W.dtype)
    src = edge_index[0].astype(jnp.int32)
    dst = edge_index[1].astype(jnp.int32)
    w = edge_weight.astype(jnp.float32)
    pad = E_PAD - E
    srcp = jnp.pad(src, (0, pad)).reshape(NTILES, NCH, CH)
    dstp = jnp.pad(dst, (0, pad)).reshape(NTILES, NCH, CH)
    wp = jnp.pad(w, (0, pad)).reshape(NTILES, NCH, CH)
    zeros = jnp.zeros((N_PAD, DH), _f32)
    be = b_emb.reshape(1, D)
    bro = b_ro.reshape(1, D)

    h, nb, ha = _tc_pro(x, W_emb, be, W_lin, A)
    for _ in range(NUM_ITERS - 1):
        agg = _sc_agg(nb.reshape(2 * N, DH), srcp, dstp, wp, zeros)
        h, nb, ha = _tc_upd(h, ha, agg.reshape(2, N_PAD, DH), W_lin, A)
    agg = _sc_agg(nb.reshape(2 * N, DH), srcp, dstp, wp, zeros)
    out = _tc_ro(h, ha, agg.reshape(2, N_PAD, DH), W_ro, bro)
    return out
